# baseline (speedup 1.0000x reference)
import functools

import jax
import jax.numpy as jnp
from jax import lax
from jax.experimental import pallas as pl
from jax.experimental.pallas import tpu as pltpu
from jax.experimental.pallas import tpu_sc as plsc

D_MODEL = 64
SCALE = 8.0
NUM_WORKERS = 32
CHUNK = 128


def _emb_body(x_hbm, table_hbm, out_hbm, idx_v, rows_v, gsem, *, per_w):
    n_chunks = per_w // CHUNK
    wid = lax.axis_index("s") * 2 + lax.axis_index("c")
    base = wid * per_w

    pltpu.sync_copy(x_hbm.at[wid], idx_v)

    @pl.loop(0, n_chunks)
    def _chunk(j):
        pltpu.async_copy(table_hbm.at[idx_v.at[j]], rows_v, gsem).wait()

        @pl.loop(0, CHUNK)
        def _row(r):
            for k in range(D_MODEL // 16):
                sl = pl.ds(k * 16, 16)
                rows_v[r, sl] = rows_v[r, sl] * SCALE

        pltpu.sync_copy(rows_v, out_hbm.at[pl.ds(base + j * CHUNK, CHUNK)])


def kernel(x, table):
    B, L = x.shape
    N = B * L
    per_w = N // NUM_WORKERS
    assert N % (NUM_WORKERS * CHUNK) == 0
    n_chunks = per_w // CHUNK

    xf = x.reshape(NUM_WORKERS, n_chunks, CHUNK).astype(jnp.int32)
    mesh = plsc.VectorSubcoreMesh(core_axis_name="c", subcore_axis_name="s")

    emb = functools.partial(
        pl.kernel,
        out_type=jax.ShapeDtypeStruct((N, D_MODEL), jnp.float32),
        mesh=mesh,
        compiler_params=pltpu.CompilerParams(use_tc_tiling_on_sc=False),
        scratch_types=[
            pltpu.VMEM((n_chunks, CHUNK), jnp.int32),
            pltpu.VMEM((CHUNK, D_MODEL), jnp.float32),
            pltpu.SemaphoreType.DMA,
        ],
    )(functools.partial(_emb_body, per_w=per_w))

    out = emb(xf, table)
    return out.reshape(B, L, D_MODEL)
